# baseline (speedup 1.0000x reference)
import functools
import math

import jax
import jax.numpy as jnp
from jax import lax
from jax.experimental import pallas as pl
from jax.experimental.pallas import tpu as pltpu

_NH = 8
_QB = 8
_NBUF = 4


def _prep_kernel(dec_ref, am_ref, maskf_ref, ln0g_ref, ln0b_ref, wq_ref,
                 bq_ref, wk_ref,
                 x_ref, qt_ref, pmat_ref, sortedk_ref, sel_ref, ms_ref,
                 chg_ref, fc_ref, *, gsteps):
    N = dec_ref.shape[0]
    D = dec_ref.shape[1]
    dh = D // _NH
    dec = dec_ref[...]
    mu = jnp.mean(dec, axis=1, keepdims=True)
    var = jnp.mean((dec - mu) ** 2, axis=1, keepdims=True)
    x = (dec - mu) / jnp.sqrt(var + 1e-5) * ln0g_ref[...] + ln0b_ref[...]
    x_ref[...] = x
    am = am_ref[...]
    Mp1 = am.shape[1]
    mx = jnp.max(am, axis=1, keepdims=True)
    col = lax.broadcasted_iota(jnp.int32, (N, Mp1), 1)
    amax = jnp.min(jnp.where(am == mx, col, Mp1), axis=1, keepdims=True)
    samples = amax - 1
    sel = (samples >= 0).astype(jnp.float32)
    kcol = jnp.maximum(samples, 0).astype(jnp.float32)
    sel_ref[...] = sel
    eye = (lax.broadcasted_iota(jnp.int32, (N, N), 0)
           == lax.broadcasted_iota(jnp.int32, (N, N), 1)).astype(jnp.float32)
    krow = lax.dot_general(kcol, eye, (((0,), (0,)), ((), ())),
                           precision=lax.Precision.HIGHEST,
                           preferred_element_type=jnp.float32)
    m_io = lax.broadcasted_iota(jnp.int32, (N, N), 0)
    n_io = lax.broadcasted_iota(jnp.int32, (N, N), 1)
    kc_b = kcol
    kr_b = krow
    less = kc_b < kr_b
    tie = (kc_b == kr_b) & (m_io < n_io)
    A = jnp.where(less | tie, 1.0, 0.0)
    ones_row = jnp.ones((1, N), jnp.float32)
    rank = jnp.dot(ones_row, A, precision=lax.Precision.HIGHEST,
                   preferred_element_type=jnp.float32)
    r_io = lax.broadcasted_iota(jnp.int32, (N, N), 0).astype(jnp.float32)
    P = jnp.where(rank == r_io, 1.0, 0.0)
    pmat_ref[...] = P
    sortedk = lax.dot_general(P, krow, (((1,), (1,)), ((), ())),
                              precision=lax.Precision.HIGHEST,
                              preferred_element_type=jnp.float32)
    sortedk_ref[...] = sortedk
    r_col = lax.broadcasted_iota(jnp.int32, (N, 1), 0)
    sub = jnp.where(n_io == m_io - 1, 1.0, 0.0)
    sk_prev = lax.dot_general(sub, sortedk, (((1,), (0,)), ((), ())),
                              precision=lax.Precision.HIGHEST,
                              preferred_element_type=jnp.float32)
    at_start = (r_col % gsteps) == 0
    chg = jnp.where((sortedk != sk_prev) | at_start, 1.0, 0.0)
    chg_ref[...] = chg
    tri = jnp.where((n_io <= m_io) & ((n_io // gsteps) == (m_io // gsteps)),
                    1.0, 0.0)
    fc_ref[...] = lax.dot_general(tri, chg, (((1,), (0,)), ((), ())),
                                  precision=lax.Precision.HIGHEST,
                                  preferred_element_type=jnp.float32)
    Ment = maskf_ref.shape[0]
    e_io = lax.broadcasted_iota(jnp.int32, (N, Ment), 1).astype(jnp.float32)
    E1 = jnp.where(sortedk == e_io, 1.0, 0.0)
    ms_ref[...] = jnp.dot(E1, maskf_ref[...],
                          preferred_element_type=jnp.float32)
    x_s = jnp.dot(P, x, precision=lax.Precision.HIGHEST,
                  preferred_element_type=jnp.float32)
    qh = jnp.dot(x_s, wq_ref[...], preferred_element_type=jnp.float32) + bq_ref[...]
    for h in range(_NH):
        qs = qh[:, h * dh:(h + 1) * dh]
        wk_h = wk_ref[:, h * dh:(h + 1) * dh]
        qt_ref[:, h, :] = lax.dot_general(
            qs, wk_h, (((1,), (1,)), ((), ())),
            preferred_element_type=jnp.float32)


def _attend_kernel(er_ref, ch_ref, sl_ref, ehbm, vhbm, ms_ref, qt_ref,
                   out_ref, ebuf, vbuf, esem, vsem, *, scale, gsteps):
    g = pl.program_id(0)

    def issue(t):
        for j in range(_QB):
            r = j * gsteps + t

            @pl.when(ch_ref[r] == 1)
            def _():
                idx = er_ref[r]
                slot = sl_ref[r]
                pltpu.make_async_copy(ehbm.at[idx], ebuf.at[slot, j],
                                      esem.at[slot, j]).start()
                pltpu.make_async_copy(vhbm.at[idx], vbuf.at[slot, j],
                                      vsem.at[slot, j]).start()

    @pl.when(g == 0)
    def _prologue():
        for t in range(_NBUF - 1):
            issue(t)

    @pl.when(g + _NBUF - 1 < gsteps)
    def _lookahead():
        issue(g + _NBUF - 1)

    for j in range(_QB):
        r = j * gsteps + g

        @pl.when(ch_ref[r] == 1)
        def _wait():
            slot = sl_ref[r]
            pltpu.make_async_copy(ehbm.at[0], ebuf.at[slot, j],
                                  esem.at[slot, j]).wait()
            pltpu.make_async_copy(vhbm.at[0], vbuf.at[slot, j],
                                  vsem.at[slot, j]).wait()
    for j in range(_QB):
        slot = sl_ref[j * gsteps + g]
        q = qt_ref[j, 0]
        e = ebuf[slot, j]
        v = vbuf[slot, j]
        scores = lax.dot_general(q, e, (((1,), (1,)), ((), ())),
                                 preferred_element_type=jnp.float32) * scale
        mb = ms_ref[j, 0]
        scores = jnp.where(mb > 0.0, scores, -1e9)
        mx = jnp.max(scores, axis=1, keepdims=True)
        ex = jnp.exp(scores - mx)
        attn = ex / jnp.sum(ex, axis=1, keepdims=True)
        out_ref[j, 0] = lax.dot_general(
            attn, v, (((1,), (0,)), ((), ())),
            preferred_element_type=jnp.float32)


def _post_kernel(ct_ref, pmat_ref, x_ref, sel_ref, wv_ref, bv_ref, wo_ref, bo_ref,
                 f1w1_ref, f1b1_ref, f1w2_ref, f1b2_ref,
                 f2w1_ref, f2b1_ref, f2w2_ref, f2b2_ref,
                 ln1g_ref, ln1b_ref, out_ref):
    D = x_ref.shape[1]
    dh = D // _NH
    parts = []
    for h in range(_NH):
        parts.append(lax.dot_general(
            ct_ref[:, h, :], wv_ref[:, h * dh:(h + 1) * dh], (((1,), (0,)), ((), ())),
            preferred_element_type=jnp.float32))
    ctx_s = jnp.concatenate(parts, axis=1) + bv_ref[...]
    ctx = lax.dot_general(pmat_ref[...], ctx_s, (((0,), (0,)), ((), ())),
                          precision=lax.Precision.HIGHEST,
                          preferred_element_type=jnp.float32)
    x = x_ref[...]
    st = jnp.dot(ctx, wo_ref[...], preferred_element_type=jnp.float32) \
        + bo_ref[...] + x
    h1 = jnp.maximum(jnp.dot(st, f1w1_ref[...],
                             preferred_element_type=jnp.float32)
                     + f1b1_ref[...], 0.0)
    st = jnp.dot(h1, f1w2_ref[...], preferred_element_type=jnp.float32) \
        + f1b2_ref[...] + st
    mu = jnp.mean(st, axis=1, keepdims=True)
    var = jnp.mean((st - mu) ** 2, axis=1, keepdims=True)
    st = (st - mu) / jnp.sqrt(var + 1e-5) * ln1g_ref[...] + ln1b_ref[...]
    xf = x + st * sel_ref[...]
    h2 = jnp.maximum(jnp.dot(xf, f2w1_ref[...],
                             preferred_element_type=jnp.float32)
                     + f2b1_ref[...], 0.0)
    out_ref[...] = jnp.dot(h2, f2w2_ref[...], preferred_element_type=jnp.float32) \
        + f2b2_ref[...] + xf


def kernel(dec_output, tgt_mask, mem_attn_out, enc_out_mem, tgt_emb_mem,
           tgt_mask_mem, ln0_g, ln0_b, ln1_g, ln1_b, Wq, bq, Wk, bk, Wv, bv,
           Wo, bo, ff1_W1, ff1_b1, ff1_W2, ff1_b2, ff2_W1, ff2_b1, ff2_W2,
           ff2_b2):
    B, L, D = dec_output.shape
    N = B * L
    M, LM, _ = enc_out_mem.shape
    dh = D // _NH
    dec = dec_output.reshape(N, D)
    am = mem_attn_out.reshape(N, M + 1)
    row = lambda a: a.reshape(1, -1)

    maskf = tgt_mask_mem.astype(jnp.float32)
    G = N // _QB
    x, qt, pmat, sortedk_f, sel_f, ms, chg_f, fc_f = pl.pallas_call(
        functools.partial(_prep_kernel, gsteps=G),
        out_shape=(
            jax.ShapeDtypeStruct((N, D), jnp.float32),
            jax.ShapeDtypeStruct((N, _NH, D), jnp.float32),
            jax.ShapeDtypeStruct((N, N), jnp.float32),
            jax.ShapeDtypeStruct((N, 1), jnp.float32),
            jax.ShapeDtypeStruct((N, 1), jnp.float32),
            jax.ShapeDtypeStruct((N, LM), jnp.float32),
            jax.ShapeDtypeStruct((N, 1), jnp.float32),
            jax.ShapeDtypeStruct((N, 1), jnp.float32),
        ),
    )(dec, am, maskf, row(ln0_g), row(ln0_b), Wq, row(bq), Wk)

    sortedk = sortedk_f.astype(jnp.int32).reshape(N)
    changed = chg_f.astype(jnp.int32).reshape(N)
    slotarr = (fc_f.astype(jnp.int32).reshape(N) - 1) % _NBUF
    qt_r = qt.reshape(_QB, G, _NH, D)
    ms_r = ms.reshape(_QB, G, 1, LM)

    ct_r = pl.pallas_call(
        functools.partial(_attend_kernel, scale=1.0 / math.sqrt(dh), gsteps=G),
        grid_spec=pltpu.PrefetchScalarGridSpec(
            num_scalar_prefetch=3,
            grid=(G,),
            in_specs=[
                pl.BlockSpec(memory_space=pl.ANY),
                pl.BlockSpec(memory_space=pl.ANY),
                pl.BlockSpec((_QB, 1, 1, LM), lambda g, er, ch, sl: (0, g, 0, 0)),
                pl.BlockSpec((_QB, 1, _NH, D), lambda g, er, ch, sl: (0, g, 0, 0)),
            ],
            out_specs=pl.BlockSpec((_QB, 1, _NH, D),
                                   lambda g, er, ch, sl: (0, g, 0, 0)),
            scratch_shapes=[
                pltpu.VMEM((_NBUF, _QB, LM, D), jnp.float32),
                pltpu.VMEM((_NBUF, _QB, LM, D), jnp.float32),
                pltpu.SemaphoreType.DMA((_NBUF, _QB)),
                pltpu.SemaphoreType.DMA((_NBUF, _QB)),
            ],
        ),
        out_shape=jax.ShapeDtypeStruct((_QB, G, _NH, D), jnp.float32),
        compiler_params=pltpu.CompilerParams(
            dimension_semantics=("arbitrary",)),
    )(sortedk, changed, slotarr, enc_out_mem, tgt_emb_mem, ms_r, qt_r)
    ct = ct_r.reshape(N, _NH, D)

    out = pl.pallas_call(
        _post_kernel,
        out_shape=jax.ShapeDtypeStruct((N, D), jnp.float32),
    )(ct, pmat, x, sel_f, Wv, row(bv), Wo, row(bo),
      ff1_W1, row(ff1_b1), ff1_W2, row(ff1_b2),
      ff2_W1, row(ff2_b1), ff2_W2, row(ff2_b2),
      row(ln1_g), row(ln1_b))

    return out.reshape(B, L, D)


# speedup vs baseline: 2.4903x; 2.4903x over previous
import functools
import math

import jax
import jax.numpy as jnp
from jax import lax
from jax.experimental import pallas as pl
from jax.experimental.pallas import tpu as pltpu

_NH = 8
_QB = 8
_NBUF = 4


def _prep_kernel(dec_ref, am_ref, maskf_ref, ln0g_ref, ln0b_ref, wq_ref,
                 bq_ref, wk_ref,
                 x_ref, qt_ref, pmat_ref, sortedk_ref, sel_ref, ms_ref):
    N = dec_ref.shape[0]
    D = dec_ref.shape[1]
    dh = D // _NH
    dec = dec_ref[...]
    mu = jnp.mean(dec, axis=1, keepdims=True)
    var = jnp.mean((dec - mu) ** 2, axis=1, keepdims=True)
    x = (dec - mu) / jnp.sqrt(var + 1e-5) * ln0g_ref[...] + ln0b_ref[...]
    x_ref[...] = x
    am = am_ref[...]
    Mp1 = am.shape[1]
    mx = jnp.max(am, axis=1, keepdims=True)
    col = lax.broadcasted_iota(jnp.int32, (N, Mp1), 1)
    amax = jnp.min(jnp.where(am == mx, col, Mp1), axis=1, keepdims=True)
    samples = amax - 1
    sel = (samples >= 0).astype(jnp.float32)
    kcol = jnp.maximum(samples, 0).astype(jnp.float32)
    sel_ref[...] = sel
    eye = (lax.broadcasted_iota(jnp.int32, (N, N), 0)
           == lax.broadcasted_iota(jnp.int32, (N, N), 1)).astype(jnp.float32)
    krow = lax.dot_general(kcol, eye, (((0,), (0,)), ((), ())),
                           precision=lax.Precision.HIGHEST,
                           preferred_element_type=jnp.float32)
    m_io = lax.broadcasted_iota(jnp.int32, (N, N), 0)
    n_io = lax.broadcasted_iota(jnp.int32, (N, N), 1)
    kc_b = kcol
    kr_b = krow
    less = kc_b < kr_b
    tie = (kc_b == kr_b) & (m_io < n_io)
    A = jnp.where(less | tie, 1.0, 0.0)
    ones_row = jnp.ones((1, N), jnp.float32)
    rank = jnp.dot(ones_row, A, precision=lax.Precision.HIGHEST,
                   preferred_element_type=jnp.float32)
    r_io = lax.broadcasted_iota(jnp.int32, (N, N), 0).astype(jnp.float32)
    P = jnp.where(rank == r_io, 1.0, 0.0)
    pmat_ref[...] = P
    sortedk = lax.dot_general(P, krow, (((1,), (1,)), ((), ())),
                              precision=lax.Precision.HIGHEST,
                              preferred_element_type=jnp.float32)
    sortedk_ref[...] = sortedk
    Ment = maskf_ref.shape[0]
    e_io = lax.broadcasted_iota(jnp.int32, (N, Ment), 1).astype(jnp.float32)
    E1 = jnp.where(sortedk == e_io, 1.0, 0.0)
    ms_ref[...] = jnp.dot(E1, maskf_ref[...],
                          preferred_element_type=jnp.float32)
    x_s = jnp.dot(P, x, precision=lax.Precision.HIGHEST,
                  preferred_element_type=jnp.float32)
    qh = jnp.dot(x_s, wq_ref[...], preferred_element_type=jnp.float32) + bq_ref[...]
    for h in range(_NH):
        qs = qh[:, h * dh:(h + 1) * dh]
        wk_h = wk_ref[:, h * dh:(h + 1) * dh]
        qt_ref[:, h, :] = lax.dot_general(
            qs, wk_h, (((1,), (1,)), ((), ())),
            preferred_element_type=jnp.float32)


def _attend_kernel(er_ref, ehbm, vhbm, ms_ref, qt_ref,
                   out_ref, ebuf, vbuf, esem, vsem, *, scale, gsteps):
    g = pl.program_id(0)

    def issue(t):
        slot = lax.rem(t, _NBUF)
        for j in range(_QB):
            idx = er_ref[j * gsteps + t]
            pltpu.make_async_copy(ehbm.at[idx], ebuf.at[slot, j],
                                  esem.at[slot, j]).start()
            pltpu.make_async_copy(vhbm.at[idx], vbuf.at[slot, j],
                                  vsem.at[slot, j]).start()

    @pl.when(g == 0)
    def _prologue():
        for t in range(_NBUF - 1):
            issue(t)

    @pl.when(g + _NBUF - 1 < gsteps)
    def _lookahead():
        issue(g + _NBUF - 1)

    slot = lax.rem(g, _NBUF)
    for j in range(_QB):
        pltpu.make_async_copy(ehbm.at[0], ebuf.at[slot, j],
                              esem.at[slot, j]).wait()
        pltpu.make_async_copy(vhbm.at[0], vbuf.at[slot, j],
                              vsem.at[slot, j]).wait()
    QBH = _QB * _NH
    KT = _QB * _NH * 16
    lm = ebuf.shape[2]
    kt = _QB * lm
    q_all = qt_ref[...].reshape(QBH, ebuf.shape[3])
    e_all = ebuf[slot].reshape(kt, ebuf.shape[3])
    v_all = vbuf[slot].reshape(kt, ebuf.shape[3])
    scores = lax.dot_general(q_all, e_all, (((1,), (1,)), ((), ())),
                             preferred_element_type=jnp.float32) * scale
    i_io = lax.broadcasted_iota(jnp.int32, (QBH, kt), 0)
    k_io = lax.broadcasted_iota(jnp.int32, (QBH, kt), 1)
    bd = (i_io // _NH) == (k_io // lm)
    msk = ms_ref[...].reshape(1, kt) > 0.0
    scores = jnp.where(bd, jnp.where(msk, scores, -1e9), -2e9)
    mx = jnp.max(scores, axis=1, keepdims=True)
    ex = jnp.exp(scores - mx)
    attn = ex / jnp.sum(ex, axis=1, keepdims=True)
    ctx = lax.dot_general(attn, v_all, (((1,), (0,)), ((), ())),
                          preferred_element_type=jnp.float32)
    out_ref[...] = ctx.reshape(_QB, 1, _NH, ebuf.shape[3])


def _post_kernel(ct_ref, pmat_ref, x_ref, sel_ref, wv_ref, bv_ref, wo_ref, bo_ref,
                 f1w1_ref, f1b1_ref, f1w2_ref, f1b2_ref,
                 f2w1_ref, f2b1_ref, f2w2_ref, f2b2_ref,
                 ln1g_ref, ln1b_ref, out_ref):
    D = x_ref.shape[1]
    dh = D // _NH
    parts = []
    for h in range(_NH):
        parts.append(lax.dot_general(
            ct_ref[:, h, :], wv_ref[:, h * dh:(h + 1) * dh], (((1,), (0,)), ((), ())),
            preferred_element_type=jnp.float32))
    ctx_s = jnp.concatenate(parts, axis=1) + bv_ref[...]
    ctx = lax.dot_general(pmat_ref[...], ctx_s, (((0,), (0,)), ((), ())),
                          precision=lax.Precision.HIGHEST,
                          preferred_element_type=jnp.float32)
    x = x_ref[...]
    st = jnp.dot(ctx, wo_ref[...], preferred_element_type=jnp.float32) \
        + bo_ref[...] + x
    h1 = jnp.maximum(jnp.dot(st, f1w1_ref[...],
                             preferred_element_type=jnp.float32)
                     + f1b1_ref[...], 0.0)
    st = jnp.dot(h1, f1w2_ref[...], preferred_element_type=jnp.float32) \
        + f1b2_ref[...] + st
    mu = jnp.mean(st, axis=1, keepdims=True)
    var = jnp.mean((st - mu) ** 2, axis=1, keepdims=True)
    st = (st - mu) / jnp.sqrt(var + 1e-5) * ln1g_ref[...] + ln1b_ref[...]
    xf = x + st * sel_ref[...]
    h2 = jnp.maximum(jnp.dot(xf, f2w1_ref[...],
                             preferred_element_type=jnp.float32)
                     + f2b1_ref[...], 0.0)
    out_ref[...] = jnp.dot(h2, f2w2_ref[...], preferred_element_type=jnp.float32) \
        + f2b2_ref[...] + xf


def kernel(dec_output, tgt_mask, mem_attn_out, enc_out_mem, tgt_emb_mem,
           tgt_mask_mem, ln0_g, ln0_b, ln1_g, ln1_b, Wq, bq, Wk, bk, Wv, bv,
           Wo, bo, ff1_W1, ff1_b1, ff1_W2, ff1_b2, ff2_W1, ff2_b1, ff2_W2,
           ff2_b2):
    B, L, D = dec_output.shape
    N = B * L
    M, LM, _ = enc_out_mem.shape
    dh = D // _NH
    dec = dec_output.reshape(N, D)
    am = mem_attn_out.reshape(N, M + 1)
    row = lambda a: a.reshape(1, -1)

    maskf = tgt_mask_mem.astype(jnp.float32)
    G = N // _QB
    x, qt, pmat, sortedk_f, sel_f, ms = pl.pallas_call(
        _prep_kernel,
        out_shape=(
            jax.ShapeDtypeStruct((N, D), jnp.float32),
            jax.ShapeDtypeStruct((N, _NH, D), jnp.float32),
            jax.ShapeDtypeStruct((N, N), jnp.float32),
            jax.ShapeDtypeStruct((N, 1), jnp.float32),
            jax.ShapeDtypeStruct((N, 1), jnp.float32),
            jax.ShapeDtypeStruct((N, LM), jnp.float32),
        ),
    )(dec, am, maskf, row(ln0_g), row(ln0_b), Wq, row(bq), Wk)

    sortedk = sortedk_f.astype(jnp.int32).reshape(N)
    qt_r = qt.reshape(_QB, G, _NH, D)
    ms_r = ms.reshape(_QB, G, 1, LM)

    ct_r = pl.pallas_call(
        functools.partial(_attend_kernel, scale=1.0 / math.sqrt(dh), gsteps=G),
        grid_spec=pltpu.PrefetchScalarGridSpec(
            num_scalar_prefetch=1,
            grid=(G,),
            in_specs=[
                pl.BlockSpec(memory_space=pl.ANY),
                pl.BlockSpec(memory_space=pl.ANY),
                pl.BlockSpec((_QB, 1, 1, LM), lambda g, er: (0, g, 0, 0)),
                pl.BlockSpec((_QB, 1, _NH, D), lambda g, er: (0, g, 0, 0)),
            ],
            out_specs=pl.BlockSpec((_QB, 1, _NH, D),
                                   lambda g, er: (0, g, 0, 0)),
            scratch_shapes=[
                pltpu.VMEM((_NBUF, _QB, LM, D), jnp.float32),
                pltpu.VMEM((_NBUF, _QB, LM, D), jnp.float32),
                pltpu.SemaphoreType.DMA((_NBUF, _QB)),
                pltpu.SemaphoreType.DMA((_NBUF, _QB)),
            ],
        ),
        out_shape=jax.ShapeDtypeStruct((_QB, G, _NH, D), jnp.float32),
        compiler_params=pltpu.CompilerParams(
            dimension_semantics=("arbitrary",)),
    )(sortedk, enc_out_mem, tgt_emb_mem, ms_r, qt_r)
    ct = ct_r.reshape(N, _NH, D)

    out = pl.pallas_call(
        _post_kernel,
        out_shape=jax.ShapeDtypeStruct((N, D), jnp.float32),
    )(ct, pmat, x, sel_f, Wv, row(bv), Wo, row(bo),
      ff1_W1, row(ff1_b1), ff1_W2, row(ff1_b2),
      ff2_W1, row(ff2_b1), ff2_W2, row(ff2_b2),
      row(ln1_g), row(ln1_b))

    return out.reshape(B, L, D)


# speedup vs baseline: 2.9650x; 1.1906x over previous
import functools
import math

import jax
import jax.numpy as jnp
from jax import lax
from jax.experimental import pallas as pl
from jax.experimental.pallas import tpu as pltpu

_NH = 8
_QB = 16
_NBUF = 4


def _prep_kernel(dec_ref, am_ref, maskf_ref, ln0g_ref, ln0b_ref, wq_ref,
                 bq_ref, wk_ref,
                 x_ref, qt_ref, pmat_ref, sortedk_ref, sel_ref, ms_ref):
    N = dec_ref.shape[0]
    D = dec_ref.shape[1]
    dh = D // _NH
    dec = dec_ref[...]
    mu = jnp.mean(dec, axis=1, keepdims=True)
    var = jnp.mean((dec - mu) ** 2, axis=1, keepdims=True)
    x = (dec - mu) / jnp.sqrt(var + 1e-5) * ln0g_ref[...] + ln0b_ref[...]
    x_ref[...] = x
    am = am_ref[...]
    Mp1 = am.shape[1]
    mx = jnp.max(am, axis=1, keepdims=True)
    col = lax.broadcasted_iota(jnp.int32, (N, Mp1), 1)
    amax = jnp.min(jnp.where(am == mx, col, Mp1), axis=1, keepdims=True)
    samples = amax - 1
    sel = (samples >= 0).astype(jnp.float32)
    kcol = jnp.maximum(samples, 0).astype(jnp.float32)
    sel_ref[...] = sel
    eye = (lax.broadcasted_iota(jnp.int32, (N, N), 0)
           == lax.broadcasted_iota(jnp.int32, (N, N), 1)).astype(jnp.float32)
    krow = lax.dot_general(kcol, eye, (((0,), (0,)), ((), ())),
                           precision=lax.Precision.HIGHEST,
                           preferred_element_type=jnp.float32)
    m_io = lax.broadcasted_iota(jnp.int32, (N, N), 0)
    n_io = lax.broadcasted_iota(jnp.int32, (N, N), 1)
    kc_b = kcol
    kr_b = krow
    less = kc_b < kr_b
    tie = (kc_b == kr_b) & (m_io < n_io)
    A = jnp.where(less | tie, 1.0, 0.0)
    ones_row = jnp.ones((1, N), jnp.float32)
    rank = jnp.dot(ones_row, A, precision=lax.Precision.HIGHEST,
                   preferred_element_type=jnp.float32)
    r_io = lax.broadcasted_iota(jnp.int32, (N, N), 0).astype(jnp.float32)
    P = jnp.where(rank == r_io, 1.0, 0.0)
    pmat_ref[...] = P
    sortedk = lax.dot_general(P, krow, (((1,), (1,)), ((), ())),
                              precision=lax.Precision.HIGHEST,
                              preferred_element_type=jnp.float32)
    sortedk_ref[...] = sortedk
    Ment = maskf_ref.shape[0]
    e_io = lax.broadcasted_iota(jnp.int32, (N, Ment), 1).astype(jnp.float32)
    E1 = jnp.where(sortedk == e_io, 1.0, 0.0)
    ms_ref[...] = jnp.dot(E1, maskf_ref[...],
                          preferred_element_type=jnp.float32)
    x_s = jnp.dot(P, x, precision=lax.Precision.HIGHEST,
                  preferred_element_type=jnp.float32)
    qh = jnp.dot(x_s, wq_ref[...], preferred_element_type=jnp.float32) + bq_ref[...]
    for h in range(_NH):
        qs = qh[:, h * dh:(h + 1) * dh]
        wk_h = wk_ref[:, h * dh:(h + 1) * dh]
        qt_ref[:, h, :] = lax.dot_general(
            qs, wk_h, (((1,), (1,)), ((), ())),
            preferred_element_type=jnp.float32)


def _attend_kernel(er_ref, ehbm, vhbm, ms_ref, qt_ref,
                   out_ref, ebuf, vbuf, esem, vsem, *, scale, gsteps):
    g = pl.program_id(0)

    def issue(t):
        slot = lax.rem(t, _NBUF)
        for j in range(_QB):
            idx = er_ref[j * gsteps + t]
            pltpu.make_async_copy(ehbm.at[idx], ebuf.at[slot, j],
                                  esem.at[slot, j]).start()
            pltpu.make_async_copy(vhbm.at[idx], vbuf.at[slot, j],
                                  vsem.at[slot, j]).start()

    @pl.when(g == 0)
    def _prologue():
        for t in range(_NBUF - 1):
            issue(t)

    @pl.when(g + _NBUF - 1 < gsteps)
    def _lookahead():
        issue(g + _NBUF - 1)

    slot = lax.rem(g, _NBUF)
    for j in range(_QB):
        pltpu.make_async_copy(ehbm.at[0], ebuf.at[slot, j],
                              esem.at[slot, j]).wait()
        pltpu.make_async_copy(vhbm.at[0], vbuf.at[slot, j],
                              vsem.at[slot, j]).wait()
    QBH = _QB * _NH
    KT = _QB * _NH * 16
    lm = ebuf.shape[2]
    kt = _QB * lm
    q_all = qt_ref[...].reshape(QBH, ebuf.shape[3])
    e_all = ebuf[slot].reshape(kt, ebuf.shape[3])
    v_all = vbuf[slot].reshape(kt, ebuf.shape[3])
    scores = lax.dot_general(q_all, e_all, (((1,), (1,)), ((), ())),
                             preferred_element_type=jnp.float32) * scale
    i_io = lax.broadcasted_iota(jnp.int32, (QBH, kt), 0)
    k_io = lax.broadcasted_iota(jnp.int32, (QBH, kt), 1)
    bd = (i_io // _NH) == (k_io // lm)
    msk = ms_ref[...].reshape(1, kt) > 0.0
    scores = jnp.where(bd, jnp.where(msk, scores, -1e9), -2e9)
    mx = jnp.max(scores, axis=1, keepdims=True)
    ex = jnp.exp(scores - mx)
    attn = ex / jnp.sum(ex, axis=1, keepdims=True)
    ctx = lax.dot_general(attn, v_all, (((1,), (0,)), ((), ())),
                          preferred_element_type=jnp.float32)
    out_ref[...] = ctx.reshape(_QB, 1, _NH, ebuf.shape[3])


def _post_kernel(ct_ref, pmat_ref, x_ref, sel_ref, wv_ref, bv_ref, wo_ref, bo_ref,
                 f1w1_ref, f1b1_ref, f1w2_ref, f1b2_ref,
                 f2w1_ref, f2b1_ref, f2w2_ref, f2b2_ref,
                 ln1g_ref, ln1b_ref, out_ref):
    D = x_ref.shape[1]
    dh = D // _NH
    parts = []
    for h in range(_NH):
        parts.append(lax.dot_general(
            ct_ref[:, h, :], wv_ref[:, h * dh:(h + 1) * dh], (((1,), (0,)), ((), ())),
            preferred_element_type=jnp.float32))
    ctx_s = jnp.concatenate(parts, axis=1) + bv_ref[...]
    ctx = lax.dot_general(pmat_ref[...], ctx_s, (((0,), (0,)), ((), ())),
                          precision=lax.Precision.HIGHEST,
                          preferred_element_type=jnp.float32)
    x = x_ref[...]
    st = jnp.dot(ctx, wo_ref[...], preferred_element_type=jnp.float32) \
        + bo_ref[...] + x
    h1 = jnp.maximum(jnp.dot(st, f1w1_ref[...],
                             preferred_element_type=jnp.float32)
                     + f1b1_ref[...], 0.0)
    st = jnp.dot(h1, f1w2_ref[...], preferred_element_type=jnp.float32) \
        + f1b2_ref[...] + st
    mu = jnp.mean(st, axis=1, keepdims=True)
    var = jnp.mean((st - mu) ** 2, axis=1, keepdims=True)
    st = (st - mu) / jnp.sqrt(var + 1e-5) * ln1g_ref[...] + ln1b_ref[...]
    xf = x + st * sel_ref[...]
    h2 = jnp.maximum(jnp.dot(xf, f2w1_ref[...],
                             preferred_element_type=jnp.float32)
                     + f2b1_ref[...], 0.0)
    out_ref[...] = jnp.dot(h2, f2w2_ref[...], preferred_element_type=jnp.float32) \
        + f2b2_ref[...] + xf


def kernel(dec_output, tgt_mask, mem_attn_out, enc_out_mem, tgt_emb_mem,
           tgt_mask_mem, ln0_g, ln0_b, ln1_g, ln1_b, Wq, bq, Wk, bk, Wv, bv,
           Wo, bo, ff1_W1, ff1_b1, ff1_W2, ff1_b2, ff2_W1, ff2_b1, ff2_W2,
           ff2_b2):
    B, L, D = dec_output.shape
    N = B * L
    M, LM, _ = enc_out_mem.shape
    dh = D // _NH
    dec = dec_output.reshape(N, D)
    am = mem_attn_out.reshape(N, M + 1)
    row = lambda a: a.reshape(1, -1)

    maskf = tgt_mask_mem.astype(jnp.float32)
    G = N // _QB
    x, qt, pmat, sortedk_f, sel_f, ms = pl.pallas_call(
        _prep_kernel,
        out_shape=(
            jax.ShapeDtypeStruct((N, D), jnp.float32),
            jax.ShapeDtypeStruct((N, _NH, D), jnp.float32),
            jax.ShapeDtypeStruct((N, N), jnp.float32),
            jax.ShapeDtypeStruct((N, 1), jnp.float32),
            jax.ShapeDtypeStruct((N, 1), jnp.float32),
            jax.ShapeDtypeStruct((N, LM), jnp.float32),
        ),
    )(dec, am, maskf, row(ln0_g), row(ln0_b), Wq, row(bq), Wk)

    sortedk = sortedk_f.astype(jnp.int32).reshape(N)
    qt_r = qt.reshape(_QB, G, _NH, D)
    ms_r = ms.reshape(_QB, G, 1, LM)

    ct_r = pl.pallas_call(
        functools.partial(_attend_kernel, scale=1.0 / math.sqrt(dh), gsteps=G),
        grid_spec=pltpu.PrefetchScalarGridSpec(
            num_scalar_prefetch=1,
            grid=(G,),
            in_specs=[
                pl.BlockSpec(memory_space=pl.ANY),
                pl.BlockSpec(memory_space=pl.ANY),
                pl.BlockSpec((_QB, 1, 1, LM), lambda g, er: (0, g, 0, 0)),
                pl.BlockSpec((_QB, 1, _NH, D), lambda g, er: (0, g, 0, 0)),
            ],
            out_specs=pl.BlockSpec((_QB, 1, _NH, D),
                                   lambda g, er: (0, g, 0, 0)),
            scratch_shapes=[
                pltpu.VMEM((_NBUF, _QB, LM, D), jnp.float32),
                pltpu.VMEM((_NBUF, _QB, LM, D), jnp.float32),
                pltpu.SemaphoreType.DMA((_NBUF, _QB)),
                pltpu.SemaphoreType.DMA((_NBUF, _QB)),
            ],
        ),
        out_shape=jax.ShapeDtypeStruct((_QB, G, _NH, D), jnp.float32),
        compiler_params=pltpu.CompilerParams(
            dimension_semantics=("arbitrary",)),
    )(sortedk, enc_out_mem, tgt_emb_mem, ms_r, qt_r)
    ct = ct_r.reshape(N, _NH, D)

    out = pl.pallas_call(
        _post_kernel,
        out_shape=jax.ShapeDtypeStruct((N, D), jnp.float32),
    )(ct, pmat, x, sel_f, Wv, row(bv), Wo, row(bo),
      ff1_W1, row(ff1_b1), ff1_W2, row(ff1_b2),
      ff2_W1, row(ff2_b1), ff2_W2, row(ff2_b2),
      row(ln1_g), row(ln1_b))

    return out.reshape(B, L, D)
